# baseline (speedup 1.0000x reference)
import jax
import jax.numpy as jnp
from jax.experimental import pallas as pl
from jax.experimental.pallas import tpu as pltpu

K = 32
D = 16
F = K * D
HID = 128
EPS = 1e-12

NODES = 10000
NN = 1024
GRID = -(-NODES // NN)


def _acos(c):
    ax = jnp.abs(c)
    p = jnp.float32(-0.0012624911)
    p = p * ax + jnp.float32(0.0066700901)
    p = p * ax + jnp.float32(-0.0170881256)
    p = p * ax + jnp.float32(0.0308918810)
    p = p * ax + jnp.float32(-0.0501743046)
    p = p * ax + jnp.float32(0.0889789874)
    p = p * ax + jnp.float32(-0.2145988016)
    p = p * ax + jnp.float32(1.5707963050)
    r = jnp.sqrt(jnp.maximum(1.0 - ax, 0.0)) * p
    return jnp.where(c >= 0, r, jnp.float32(3.14159265358979) - r)


def _fused_kernel(t_ref, w1_ref, b1_ref, w2_ref, b2_ref,
                  w3_ref, b3_ref, w4_ref, b4_ref, o_ref):
    t = t_ref[...]
    tr = jnp.roll(t, -D, axis=1)
    sel = (jax.lax.broadcasted_iota(jnp.int32, (F, K), 0) // D ==
           jax.lax.broadcasted_iota(jnp.int32, (F, K), 1)).astype(jnp.float32)
    sq = jnp.dot(t * t, sel,
                 preferred_element_type=jnp.float32) + EPS
    dt = jnp.dot(t * tr, sel,
                 preferred_element_type=jnp.float32)
    sq2 = jnp.roll(sq, -1, axis=1)
    c = dt * jax.lax.rsqrt(sq * sq2)
    c = jnp.clip(c, -1.0, 1.0)
    ang = _acos(c)
    sel2 = (jax.lax.broadcasted_iota(jnp.int32, (K, D), 0) ==
            2 * jax.lax.broadcasted_iota(jnp.int32, (K, D), 1)
            ).astype(jnp.float32)
    w1e = jnp.dot(sel2, w1_ref[...], preferred_element_type=jnp.float32)
    h = jnp.tanh(jnp.dot(ang, w1e,
                         preferred_element_type=jnp.float32) + b1_ref[...])
    h = jnp.tanh(jnp.dot(h, w2_ref[...],
                         preferred_element_type=jnp.float32) + b2_ref[...])
    h = jnp.tanh(jnp.dot(h, w3_ref[...],
                         preferred_element_type=jnp.float32) + b3_ref[...])
    o = jax.nn.sigmoid(jnp.dot(h, w4_ref[...],
                               preferred_element_type=jnp.float32) + b4_ref[...])
    o_ref[...] = o


def kernel(x, edge_index, edge_attr, W1, b1, W2, b2, W3, b3, W4, b4):
    del x, edge_index
    ea = edge_attr.reshape(NODES, F)
    out = pl.pallas_call(
        _fused_kernel,
        grid=(GRID,),
        in_specs=[
            pl.BlockSpec((NN, F), lambda i: (i, 0)),
            pl.BlockSpec((D, HID), lambda i: (0, 0)),
            pl.BlockSpec((1, HID), lambda i: (0, 0)),
            pl.BlockSpec((HID, HID), lambda i: (0, 0)),
            pl.BlockSpec((1, HID), lambda i: (0, 0)),
            pl.BlockSpec((HID, HID), lambda i: (0, 0)),
            pl.BlockSpec((1, HID), lambda i: (0, 0)),
            pl.BlockSpec((HID, 1), lambda i: (0, 0)),
            pl.BlockSpec((1, 1), lambda i: (0, 0)),
        ],
        out_specs=pl.BlockSpec((NN, 1), lambda i: (i, 0)),
        out_shape=jax.ShapeDtypeStruct((NODES, 1), jnp.float32),
        compiler_params=pltpu.CompilerParams(
            allow_input_fusion=[True, False, False, False, False, False,
                                False, False]),
    )(ea, W1, b1.reshape(1, HID), W2, b2.reshape(1, HID),
      W3, b3.reshape(1, HID), W4, b4.reshape(1, 1))
    return out[:, 0]


# speedup vs baseline: 1.0224x; 1.0224x over previous
import jax
import jax.numpy as jnp
from jax.experimental import pallas as pl
from jax.experimental.pallas import tpu as pltpu

K = 32
D = 16
F = K * D
HID = 128
EPS = 1e-12

NODES = 10000
NN = 1024
GRID = -(-NODES // NN)


def _acos(c):
    ax = jnp.abs(c)
    p = jnp.float32(-0.0012624911)
    p = p * ax + jnp.float32(0.0066700901)
    p = p * ax + jnp.float32(-0.0170881256)
    p = p * ax + jnp.float32(0.0308918810)
    p = p * ax + jnp.float32(-0.0501743046)
    p = p * ax + jnp.float32(0.0889789874)
    p = p * ax + jnp.float32(-0.2145988016)
    p = p * ax + jnp.float32(1.5707963050)
    r = jnp.sqrt(jnp.maximum(1.0 - ax, 0.0)) * p
    return jnp.where(c >= 0, r, jnp.float32(3.14159265358979) - r)


def _fused_kernel(t_ref, w1_ref, b1_ref, w2_ref, b2_ref,
                  w3_ref, b3_ref, w4_ref, b4_ref, o_ref):
    t = t_ref[...]
    tr = jnp.roll(t, -D, axis=1)
    sel = (jax.lax.broadcasted_iota(jnp.int32, (F, K), 0) // D ==
           jax.lax.broadcasted_iota(jnp.int32, (F, K), 1)).astype(jnp.float32)
    sq = jnp.dot(t * t, sel,
                 preferred_element_type=jnp.float32) + EPS
    dt = jnp.dot(t * tr, sel,
                 preferred_element_type=jnp.float32)
    sq2 = jnp.roll(sq, -1, axis=1)
    c = dt * jax.lax.rsqrt(sq * sq2)
    c = jnp.clip(c, -1.0, 1.0)
    ang = _acos(c)
    sel2 = (jax.lax.broadcasted_iota(jnp.int32, (K, D), 0) ==
            2 * jax.lax.broadcasted_iota(jnp.int32, (K, D), 1)
            ).astype(jnp.float32)
    w1e = jnp.dot(sel2, w1_ref[...], preferred_element_type=jnp.float32)
    h = jnp.tanh(jnp.dot(ang, w1e,
                         preferred_element_type=jnp.float32) + b1_ref[...])
    h = jnp.tanh(jnp.dot(h, w2_ref[...],
                         preferred_element_type=jnp.float32) + b2_ref[...])
    h = jnp.tanh(jnp.dot(h, w3_ref[...],
                         preferred_element_type=jnp.float32) + b3_ref[...])
    o = jax.nn.sigmoid(jnp.dot(h, w4_ref[...],
                               preferred_element_type=jnp.float32) + b4_ref[...])
    o_ref[...] = o.T


def kernel(x, edge_index, edge_attr, W1, b1, W2, b2, W3, b3, W4, b4):
    del x, edge_index
    ea = edge_attr.reshape(NODES, F)
    out = pl.pallas_call(
        _fused_kernel,
        grid=(GRID,),
        in_specs=[
            pl.BlockSpec((NN, F), lambda i: (i, 0)),
            pl.BlockSpec((D, HID), lambda i: (0, 0)),
            pl.BlockSpec((1, HID), lambda i: (0, 0)),
            pl.BlockSpec((HID, HID), lambda i: (0, 0)),
            pl.BlockSpec((1, HID), lambda i: (0, 0)),
            pl.BlockSpec((HID, HID), lambda i: (0, 0)),
            pl.BlockSpec((1, HID), lambda i: (0, 0)),
            pl.BlockSpec((HID, 1), lambda i: (0, 0)),
            pl.BlockSpec((1, 1), lambda i: (0, 0)),
        ],
        out_specs=pl.BlockSpec((1, NN), lambda i: (0, i)),
        out_shape=jax.ShapeDtypeStruct((1, GRID * NN), jnp.float32),
        compiler_params=pltpu.CompilerParams(
            allow_input_fusion=[True, False, False, False, False, False,
                                False, False]),
    )(ea, W1, b1.reshape(1, HID), W2, b2.reshape(1, HID),
      W3, b3.reshape(1, HID), W4, b4.reshape(1, 1))
    return out[0, :NODES]


# speedup vs baseline: 1.0330x; 1.0104x over previous
import jax
import jax.numpy as jnp
from jax.experimental import pallas as pl
from jax.experimental.pallas import tpu as pltpu

K = 32
D = 16
F = K * D
HID = 128
EPS = 1e-12

NODES = 10000
NN = 2048
GRID = -(-NODES // NN)


def _acos(c):
    ax = jnp.abs(c)
    p = jnp.float32(-0.0012624911)
    p = p * ax + jnp.float32(0.0066700901)
    p = p * ax + jnp.float32(-0.0170881256)
    p = p * ax + jnp.float32(0.0308918810)
    p = p * ax + jnp.float32(-0.0501743046)
    p = p * ax + jnp.float32(0.0889789874)
    p = p * ax + jnp.float32(-0.2145988016)
    p = p * ax + jnp.float32(1.5707963050)
    r = jnp.sqrt(jnp.maximum(1.0 - ax, 0.0)) * p
    return jnp.where(c >= 0, r, jnp.float32(3.14159265358979) - r)


def _fused_kernel(t_ref, w1_ref, b1_ref, w2_ref, b2_ref,
                  w3_ref, b3_ref, w4_ref, b4_ref, o_ref):
    t = t_ref[...]
    tr = jnp.roll(t, -D, axis=1)
    sel = (jax.lax.broadcasted_iota(jnp.int32, (F, K), 0) // D ==
           jax.lax.broadcasted_iota(jnp.int32, (F, K), 1)).astype(jnp.float32)
    sq = jnp.dot(t * t, sel,
                 preferred_element_type=jnp.float32) + EPS
    dt = jnp.dot(t * tr, sel,
                 preferred_element_type=jnp.float32)
    sq2 = jnp.roll(sq, -1, axis=1)
    c = dt * jax.lax.rsqrt(sq * sq2)
    c = jnp.clip(c, -1.0, 1.0)
    ang = _acos(c)
    sel2 = (jax.lax.broadcasted_iota(jnp.int32, (K, D), 0) ==
            2 * jax.lax.broadcasted_iota(jnp.int32, (K, D), 1)
            ).astype(jnp.float32)
    w1e = jnp.dot(sel2, w1_ref[...], preferred_element_type=jnp.float32)
    h = jnp.tanh(jnp.dot(ang, w1e,
                         preferred_element_type=jnp.float32) + b1_ref[...])
    h = jnp.tanh(jnp.dot(h, w2_ref[...],
                         preferred_element_type=jnp.float32) + b2_ref[...])
    h = jnp.tanh(jnp.dot(h, w3_ref[...],
                         preferred_element_type=jnp.float32) + b3_ref[...])
    o = jax.nn.sigmoid(jnp.dot(h, w4_ref[...],
                               preferred_element_type=jnp.float32) + b4_ref[...])
    o_ref[...] = o.T


def kernel(x, edge_index, edge_attr, W1, b1, W2, b2, W3, b3, W4, b4):
    del x, edge_index
    ea = edge_attr.reshape(NODES, F)
    out = pl.pallas_call(
        _fused_kernel,
        grid=(GRID,),
        in_specs=[
            pl.BlockSpec((NN, F), lambda i: (i, 0)),
            pl.BlockSpec((D, HID), lambda i: (0, 0)),
            pl.BlockSpec((1, HID), lambda i: (0, 0)),
            pl.BlockSpec((HID, HID), lambda i: (0, 0)),
            pl.BlockSpec((1, HID), lambda i: (0, 0)),
            pl.BlockSpec((HID, HID), lambda i: (0, 0)),
            pl.BlockSpec((1, HID), lambda i: (0, 0)),
            pl.BlockSpec((HID, 1), lambda i: (0, 0)),
            pl.BlockSpec((1, 1), lambda i: (0, 0)),
        ],
        out_specs=pl.BlockSpec((1, NN), lambda i: (0, i)),
        out_shape=jax.ShapeDtypeStruct((1, GRID * NN), jnp.float32),
        compiler_params=pltpu.CompilerParams(
            allow_input_fusion=[True, False, False, False, False, False,
                                False, False]),
    )(ea, W1, b1.reshape(1, HID), W2, b2.reshape(1, HID),
      W3, b3.reshape(1, HID), W4, b4.reshape(1, 1))
    return out[0, :NODES]
